# Initial kernel scaffold; baseline (speedup 1.0000x reference)
import jax
import jax.numpy as jnp
from jax.experimental import pallas as pl


def kernel(ranks, suits, rank_table, suit_table):
    raise NotImplementedError("write your pallas kernel here")


# speedup vs baseline: 9.9704x; 9.9704x over previous
import functools

import jax
import jax.numpy as jnp
from jax import lax
from jax.experimental import pallas as pl
from jax.experimental.pallas import tpu as pltpu
from jax.experimental.pallas import tpu_sc as plsc

_LANES = 16
_NUM_WORKERS = 32
_CHUNK = 3200


def _card_embed_body(ranks_hbm, suits_hbm, rtab_hbm, stab_hbm, out_hbm,
                     rtab_v, stab_v, ranks_v, suits_v, out_v, per_worker):
    num_cores = jax.lax.axis_size("c")
    wid = lax.axis_index("s") * num_cores + lax.axis_index("c")

    pltpu.sync_copy(rtab_hbm, rtab_v)
    pltpu.sync_copy(stab_hbm, stab_v)

    lanes12 = lax.iota(jnp.int32, _LANES) * 12

    def do_chunk(k, carry):
        base = wid * per_worker + k * _CHUNK
        pltpu.sync_copy(ranks_hbm.at[pl.ds(base, _CHUNK)], ranks_v)
        pltpu.sync_copy(suits_hbm.at[pl.ds(base, _CHUNK)], suits_v)

        def body(i, carry2):
            off = i * _LANES
            rows12 = off * 12 + lanes12
            r8 = ranks_v[pl.ds(off, _LANES)] * 8
            s4 = suits_v[pl.ds(off, _LANES)] * 4
            for c in range(8):
                vals = plsc.load_gather(rtab_v, [r8 + c])
                plsc.store_scatter(out_v, [rows12 + c], vals)
            for c in range(4):
                vals = plsc.load_gather(stab_v, [s4 + c])
                plsc.store_scatter(out_v, [rows12 + (8 + c)], vals)
            return carry2

        lax.fori_loop(0, _CHUNK // _LANES, body, 0, unroll=4)
        pltpu.sync_copy(out_v, out_hbm.at[pl.ds(base * 12, _CHUNK * 12)])
        return carry

    lax.fori_loop(0, per_worker // _CHUNK, do_chunk, 0)


@jax.jit
def kernel(ranks, suits, rank_table, suit_table):
    B, L = ranks.shape
    n = B * L
    per_worker = n // _NUM_WORKERS
    assert per_worker * _NUM_WORKERS == n and per_worker % _CHUNK == 0

    mesh = plsc.VectorSubcoreMesh(core_axis_name="c", subcore_axis_name="s")
    out = pl.kernel(
        functools.partial(_card_embed_body, per_worker=per_worker),
        out_type=jax.ShapeDtypeStruct((n * 12,), jnp.float32),
        mesh=mesh,
        compiler_params=pltpu.CompilerParams(needs_layout_passes=False),
        scratch_types=[
            pltpu.VMEM((14 * 8,), jnp.float32),
            pltpu.VMEM((5 * 4,), jnp.float32),
            pltpu.VMEM((_CHUNK,), jnp.int32),
            pltpu.VMEM((_CHUNK,), jnp.int32),
            pltpu.VMEM((_CHUNK * 12,), jnp.float32),
        ],
    )(ranks.reshape(n), suits.reshape(n),
      rank_table.reshape(14 * 8), suit_table.reshape(5 * 4))
    return out.reshape(B, L, 12)


# speedup vs baseline: 11.3979x; 1.1432x over previous
import functools

import jax
import jax.numpy as jnp
from jax import lax
from jax.experimental import pallas as pl
from jax.experimental.pallas import tpu as pltpu
from jax.experimental.pallas import tpu_sc as plsc

_LANES = 16
_NUM_WORKERS = 32
_CHUNK = 3200


def _card_embed_body(ranks_hbm, suits_hbm, rtab_hbm, stab_hbm, out_hbm,
                     rtab_v, stab_v, ranks_v, suits_v, out_v, per_worker):
    num_cores = jax.lax.axis_size("c")
    wid = lax.axis_index("s") * num_cores + lax.axis_index("c")

    pltpu.sync_copy(rtab_hbm, rtab_v)
    pltpu.sync_copy(stab_hbm, stab_v)

    lanes12 = lax.iota(jnp.int32, _LANES) * 12

    for k in range(per_worker // _CHUNK):
        base = wid * per_worker + k * _CHUNK
        pltpu.sync_copy(ranks_hbm.at[pl.ds(base, _CHUNK)], ranks_v)
        pltpu.sync_copy(suits_hbm.at[pl.ds(base, _CHUNK)], suits_v)

        @plsc.parallel_loop(0, _CHUNK, step=_LANES, unroll=4)
        def body(off):
            rows12 = off * 12 + lanes12
            r8 = ranks_v[pl.ds(off, _LANES)] * 8
            s4 = suits_v[pl.ds(off, _LANES)] * 4
            for c in range(8):
                vals = plsc.load_gather(rtab_v, [r8 + c])
                plsc.store_scatter(out_v, [rows12 + c], vals)
            for c in range(4):
                vals = plsc.load_gather(stab_v, [s4 + c])
                plsc.store_scatter(out_v, [rows12 + (8 + c)], vals)

        pltpu.sync_copy(out_v, out_hbm.at[pl.ds(base * 12, _CHUNK * 12)])


@jax.jit
def kernel(ranks, suits, rank_table, suit_table):
    B, L = ranks.shape
    n = B * L
    per_worker = n // _NUM_WORKERS
    assert per_worker * _NUM_WORKERS == n and per_worker % _CHUNK == 0

    mesh = plsc.VectorSubcoreMesh(core_axis_name="c", subcore_axis_name="s")
    out = pl.kernel(
        functools.partial(_card_embed_body, per_worker=per_worker),
        out_type=jax.ShapeDtypeStruct((n * 12,), jnp.float32),
        mesh=mesh,
        compiler_params=pltpu.CompilerParams(needs_layout_passes=False),
        scratch_types=[
            pltpu.VMEM((14 * 8,), jnp.float32),
            pltpu.VMEM((5 * 4,), jnp.float32),
            pltpu.VMEM((_CHUNK,), jnp.int32),
            pltpu.VMEM((_CHUNK,), jnp.int32),
            pltpu.VMEM((_CHUNK * 12,), jnp.float32),
        ],
    )(ranks.reshape(n), suits.reshape(n),
      rank_table.reshape(14 * 8), suit_table.reshape(5 * 4))
    return out.reshape(B, L, 12)


# speedup vs baseline: 46.6343x; 4.0915x over previous
import functools

import jax
import jax.numpy as jnp
from jax import lax
from jax.experimental import pallas as pl
from jax.experimental.pallas import tpu as pltpu
from jax.experimental.pallas import tpu_sc as plsc

_LANES = 16
_NUM_WORKERS = 32
_ICHUNK = 128
_L = 50
_D = 12


def _card_embed_body(ranks_hbm, suits_hbm, rtab_hbm, stab_hbm, out_hbm,
                     rtab_v, stab_v, ctab_v, ranks_v, suits_v, out_v, n_i):
    num_cores = jax.lax.axis_size("c")
    wid = lax.axis_index("s") * num_cores + lax.axis_index("c")

    pltpu.sync_copy(rtab_hbm, rtab_v)
    pltpu.sync_copy(stab_hbm, stab_v)

    lanes = lax.iota(jnp.int32, _LANES)

    for k in range(5):
        t = jnp.minimum(lanes + _LANES * k, 69)
        r = t // 5
        s = t - r * 5
        for c in range(_D):
            if c < 8:
                vals = plsc.load_gather(rtab_v, [r * 8 + c])
            else:
                vals = plsc.load_gather(stab_v, [s * 4 + (c - 8)])
            plsc.store_scatter(ctab_v, [t * _D + c], vals)

    lanes_l = lanes * _L
    per_w = n_i // _NUM_WORKERS
    groups_per_j = _ICHUNK // _LANES

    for kc in range(per_w // _ICHUNK):
        i0 = wid * per_w + kc * _ICHUNK
        pltpu.sync_copy(ranks_hbm.at[pl.ds(i0 * _L, _ICHUNK * _L)], ranks_v)
        pltpu.sync_copy(suits_hbm.at[pl.ds(i0 * _L, _ICHUNK * _L)], suits_v)

        @plsc.parallel_loop(0, _L * groups_per_j, step=1, unroll=4)
        def grp(g2):
            j = lax.shift_right_logical(g2, 3)
            g = lax.bitwise_and(g2, 7)
            idx = lanes_l + (g * (_LANES * _L) + j)
            r16 = plsc.load_gather(ranks_v, [idx])
            s16 = plsc.load_gather(suits_v, [idx])
            cidx = r16 * (5 * _D) + s16 * _D
            col = g * _LANES
            for c in range(_D):
                vals = plsc.load_gather(ctab_v, [cidx + c])
                out_v[c * _L + j, pl.ds(col, _LANES)] = vals

        pltpu.sync_copy(out_v, out_hbm.at[:, pl.ds(i0, _ICHUNK)])


@jax.jit
def kernel(ranks, suits, rank_table, suit_table):
    B, L = ranks.shape
    n = B * L
    per_w = B // _NUM_WORKERS
    assert per_w * _NUM_WORKERS == B and per_w % _ICHUNK == 0 and L == _L

    mesh = plsc.VectorSubcoreMesh(core_axis_name="c", subcore_axis_name="s")
    res = pl.kernel(
        functools.partial(_card_embed_body, n_i=B),
        out_type=jax.ShapeDtypeStruct((_D * _L, B), jnp.float32),
        mesh=mesh,
        compiler_params=pltpu.CompilerParams(needs_layout_passes=False),
        scratch_types=[
            pltpu.VMEM((14 * 8,), jnp.float32),
            pltpu.VMEM((5 * 4,), jnp.float32),
            pltpu.VMEM((70 * _D,), jnp.float32),
            pltpu.VMEM((_ICHUNK * _L,), jnp.int32),
            pltpu.VMEM((_ICHUNK * _L,), jnp.int32),
            pltpu.VMEM((_D * _L, _ICHUNK), jnp.float32),
        ],
    )(ranks.reshape(n), suits.reshape(n),
      rank_table.reshape(14 * 8), suit_table.reshape(5 * 4))
    return res.reshape(_D, _L, B).transpose(2, 1, 0)


# speedup vs baseline: 60.9993x; 1.3080x over previous
import functools

import jax
import jax.numpy as jnp
from jax import lax
from jax.experimental import pallas as pl
from jax.experimental.pallas import tpu as pltpu
from jax.experimental.pallas import tpu_sc as plsc

_LANES = 16
_NUM_WORKERS = 32
_ICHUNK = 128
_L = 50
_D = 12


def _card_embed_body(ranks_hbm, suits_hbm, rtab_hbm, stab_hbm, out_hbm,
                     rtab_v, stab_v, ctab_v, ranks_v, suits_v, out_v, n_i):
    num_cores = jax.lax.axis_size("c")
    wid = lax.axis_index("s") * num_cores + lax.axis_index("c")

    pltpu.sync_copy(rtab_hbm, rtab_v)
    pltpu.sync_copy(stab_hbm, stab_v)

    lanes = lax.iota(jnp.int32, _LANES)

    for k in range(5):
        t = jnp.minimum(lanes + _LANES * k, 69)
        r = t // 5
        s = t - r * 5
        for c in range(_D):
            if c < 8:
                vals = plsc.load_gather(rtab_v, [r * 8 + c])
            else:
                vals = plsc.load_gather(stab_v, [s * 4 + (c - 8)])
            plsc.store_scatter(ctab_v, [t * _D + c], vals)

    lanes_l = lanes * _L
    per_w = n_i // _NUM_WORKERS
    groups_per_j = _ICHUNK // _LANES

    for kc in range(per_w // _ICHUNK):
        i0 = wid * per_w + kc * _ICHUNK
        pltpu.sync_copy(ranks_hbm.at[pl.ds(i0 * _L, _ICHUNK * _L)], ranks_v)
        pltpu.sync_copy(suits_hbm.at[pl.ds(i0 * _L, _ICHUNK * _L)], suits_v)

        @plsc.parallel_loop(0, _L * groups_per_j, step=1, unroll=4)
        def grp(g2):
            j = lax.shift_right_logical(g2, 3)
            g = lax.bitwise_and(g2, 7)
            idx = lanes_l + (g * (_LANES * _L) + j)
            r16 = plsc.load_gather(ranks_v, [idx])
            s16 = plsc.load_gather(suits_v, [idx])
            cidx = r16 * (5 * _D) + s16 * _D
            col = g * _LANES
            for c in range(_D):
                vals = plsc.load_gather(ctab_v, [cidx + c])
                out_v[c * _L + j, pl.ds(col, _LANES)] = vals

        for c in range(_D):
            pltpu.sync_copy(out_v.at[pl.ds(c * _L, _L), :],
                            out_hbm.at[c, :, pl.ds(i0, _ICHUNK)])


@jax.jit
def kernel(ranks, suits, rank_table, suit_table):
    B, L = ranks.shape
    n = B * L
    per_w = B // _NUM_WORKERS
    assert per_w * _NUM_WORKERS == B and per_w % _ICHUNK == 0 and L == _L

    mesh = plsc.VectorSubcoreMesh(core_axis_name="c", subcore_axis_name="s")
    res = pl.kernel(
        functools.partial(_card_embed_body, n_i=B),
        out_type=jax.ShapeDtypeStruct((_D, _L, B), jnp.float32),
        mesh=mesh,
        compiler_params=pltpu.CompilerParams(needs_layout_passes=False),
        scratch_types=[
            pltpu.VMEM((14 * 8,), jnp.float32),
            pltpu.VMEM((5 * 4,), jnp.float32),
            pltpu.VMEM((70 * _D,), jnp.float32),
            pltpu.VMEM((_ICHUNK * _L,), jnp.int32),
            pltpu.VMEM((_ICHUNK * _L,), jnp.int32),
            pltpu.VMEM((_D * _L, _ICHUNK), jnp.float32),
        ],
    )(ranks.reshape(n), suits.reshape(n),
      rank_table.reshape(14 * 8), suit_table.reshape(5 * 4))
    return res.transpose(2, 1, 0)


# speedup vs baseline: 61.3931x; 1.0065x over previous
import functools

import jax
import jax.numpy as jnp
from jax import lax
from jax.experimental import pallas as pl
from jax.experimental.pallas import tpu as pltpu
from jax.experimental.pallas import tpu_sc as plsc

_LANES = 16
_NUM_WORKERS = 32
_ICHUNK = 128
_L = 50
_D = 12


def _card_embed_body(ranks_hbm, suits_hbm, rtab_hbm, stab_hbm, out_hbm,
                     rtab_v, stab_v, ctab_v, ranks_v, suits_v, out_v, n_i):
    num_cores = jax.lax.axis_size("c")
    wid = lax.axis_index("s") * num_cores + lax.axis_index("c")

    pltpu.sync_copy(rtab_hbm, rtab_v)
    pltpu.sync_copy(stab_hbm, stab_v)

    lanes = lax.iota(jnp.int32, _LANES)

    for k in range(5):
        t = jnp.minimum(lanes + _LANES * k, 69)
        r = t // 5
        s = t - r * 5
        for c in range(_D):
            if c < 8:
                vals = plsc.load_gather(rtab_v, [r * 8 + c])
            else:
                vals = plsc.load_gather(stab_v, [s * 4 + (c - 8)])
            plsc.store_scatter(ctab_v, [t * _D + c], vals)

    per_w = n_i // _NUM_WORKERS
    groups_per_j = _ICHUNK // _LANES

    for kc in range(per_w // _ICHUNK):
        i0 = wid * per_w + kc * _ICHUNK
        pltpu.sync_copy(ranks_hbm.at[:, pl.ds(i0, _ICHUNK)], ranks_v)
        pltpu.sync_copy(suits_hbm.at[:, pl.ds(i0, _ICHUNK)], suits_v)

        @plsc.parallel_loop(0, _L * groups_per_j, step=1, unroll=4)
        def grp(g2):
            j = lax.shift_right_logical(g2, 3)
            g = lax.bitwise_and(g2, 7)
            iv = lanes + g * _LANES
            jv = jnp.full((_LANES,), 0, jnp.int32) + j
            r16 = plsc.load_gather(ranks_v, [jv, iv])
            s16 = plsc.load_gather(suits_v, [jv, iv])
            cidx = r16 * (5 * _D) + s16 * _D
            col = g * _LANES
            for c in range(_D):
                vals = plsc.load_gather(ctab_v, [cidx + c])
                out_v[c * _L + j, pl.ds(col, _LANES)] = vals

        for c in range(_D):
            pltpu.sync_copy(out_v.at[pl.ds(c * _L, _L), :],
                            out_hbm.at[c, :, pl.ds(i0, _ICHUNK)])


@jax.jit
def kernel(ranks, suits, rank_table, suit_table):
    B, L = ranks.shape
    per_w = B // _NUM_WORKERS
    assert per_w * _NUM_WORKERS == B and per_w % _ICHUNK == 0 and L == _L

    mesh = plsc.VectorSubcoreMesh(core_axis_name="c", subcore_axis_name="s")
    res = pl.kernel(
        functools.partial(_card_embed_body, n_i=B),
        out_type=jax.ShapeDtypeStruct((_D, _L, B), jnp.float32),
        mesh=mesh,
        compiler_params=pltpu.CompilerParams(needs_layout_passes=False),
        scratch_types=[
            pltpu.VMEM((14 * 8,), jnp.float32),
            pltpu.VMEM((5 * 4,), jnp.float32),
            pltpu.VMEM((70 * _D,), jnp.float32),
            pltpu.VMEM((_L, _ICHUNK), jnp.int32),
            pltpu.VMEM((_L, _ICHUNK), jnp.int32),
            pltpu.VMEM((_D * _L, _ICHUNK), jnp.float32),
        ],
    )(ranks.T, suits.T, rank_table.reshape(14 * 8), suit_table.reshape(5 * 4))
    return res.transpose(2, 1, 0)


# speedup vs baseline: 72.7008x; 1.1842x over previous
import functools

import jax
import jax.numpy as jnp
from jax import lax
from jax.experimental import pallas as pl
from jax.experimental.pallas import tpu as pltpu
from jax.experimental.pallas import tpu_sc as plsc

_LANES = 16
_NUM_WORKERS = 32
_ICHUNK = 128
_L = 50
_LPAD = 56
_JSPLIT = 24
_D = 12


def _card_embed_body(ranks_hbm, suits_hbm, rtab_hbm, stab_hbm, out_hbm,
                     rtab_v, stab_v, ctab_v, rk0, rk1, st0, st1, oa, ob,
                     sin0, sin1, sa, sb, n_i):
    num_cores = jax.lax.axis_size("c")
    wid = lax.axis_index("s") * num_cores + lax.axis_index("c")

    pltpu.sync_copy(rtab_hbm, rtab_v)
    pltpu.sync_copy(stab_hbm, stab_v)

    lanes = lax.iota(jnp.int32, _LANES)

    for k in range(5):
        t = jnp.minimum(lanes + _LANES * k, 69)
        r = t // 5
        s = t - r * 5
        for c in range(_D):
            if c < 8:
                vals = plsc.load_gather(rtab_v, [r * 8 + c])
            else:
                vals = plsc.load_gather(stab_v, [s * 4 + (c - 8)])
            plsc.store_scatter(ctab_v, [t * _D + c], vals)

    per_w = n_i // _NUM_WORKERS
    groups_per_j = _ICHUNK // _LANES
    nchunks = per_w // _ICHUNK

    def start_in(kc, rk, st, sem):
        i0 = wid * per_w + kc * _ICHUNK
        return (
            pltpu.async_copy(ranks_hbm.at[:, pl.ds(i0, _ICHUNK)], rk, sem),
            pltpu.async_copy(suits_hbm.at[:, pl.ds(i0, _ICHUNK)], st, sem),
        )

    def half(kc, j0, nj, obuf, sem, rk, st):
        i0 = wid * per_w + kc * _ICHUNK

        @plsc.parallel_loop(0, nj * groups_per_j, step=1, unroll=4)
        def grp(g2):
            jl = lax.shift_right_logical(g2, 3)
            g = lax.bitwise_and(g2, 7)
            col = g * _LANES
            j = jnp.minimum(j0 + jl, _L - 1)
            r16 = rk[j, pl.ds(col, _LANES)]
            s16 = st[j, pl.ds(col, _LANES)]
            cidx = r16 * (5 * _D) + s16 * _D
            for c in range(_D):
                vals = plsc.load_gather(ctab_v, [cidx + c])
                obuf[c * nj + jl, pl.ds(col, _LANES)] = vals

        return [
            pltpu.async_copy(obuf.at[pl.ds(c * nj, nj), :],
                             out_hbm.at[c, pl.ds(j0, nj), pl.ds(i0, _ICHUNK)],
                             sem)
            for c in range(_D)
        ]

    in_bufs = ((rk0, st0, sin0), (rk1, st1, sin1))
    in_handles = {0: start_in(0, *in_bufs[0])}
    pend = {"a": None, "b": None}

    for kc in range(nchunks):
        rk, st, _ = in_bufs[kc % 2]
        for h in in_handles.pop(kc):
            h.wait()
        if kc + 1 < nchunks:
            in_handles[kc + 1] = start_in(kc + 1, *in_bufs[(kc + 1) % 2])
        for key, j0, nj, obuf, sem in (
                ("a", 0, _JSPLIT, oa, sa),
                ("b", _JSPLIT, _LPAD - _JSPLIT, ob, sb)):
            if pend[key] is not None:
                for h in pend[key]:
                    h.wait()
            pend[key] = half(kc, j0, nj, obuf, sem, rk, st)

    for key in ("a", "b"):
        for h in pend[key]:
            h.wait()


@jax.jit
def kernel(ranks, suits, rank_table, suit_table):
    B, L = ranks.shape
    per_w = B // _NUM_WORKERS
    assert per_w * _NUM_WORKERS == B and per_w % _ICHUNK == 0 and L == _L

    mesh = plsc.VectorSubcoreMesh(core_axis_name="c", subcore_axis_name="s")
    res = pl.kernel(
        functools.partial(_card_embed_body, n_i=B),
        out_type=jax.ShapeDtypeStruct((_D, _LPAD, B), jnp.float32),
        mesh=mesh,
        compiler_params=pltpu.CompilerParams(needs_layout_passes=False),
        scratch_types=[
            pltpu.VMEM((14 * 8,), jnp.float32),
            pltpu.VMEM((5 * 4,), jnp.float32),
            pltpu.VMEM((70 * _D,), jnp.float32),
            pltpu.VMEM((_L, _ICHUNK), jnp.int32),
            pltpu.VMEM((_L, _ICHUNK), jnp.int32),
            pltpu.VMEM((_L, _ICHUNK), jnp.int32),
            pltpu.VMEM((_L, _ICHUNK), jnp.int32),
            pltpu.VMEM((_D * _JSPLIT, _ICHUNK), jnp.float32),
            pltpu.VMEM((_D * (_LPAD - _JSPLIT), _ICHUNK), jnp.float32),
            pltpu.SemaphoreType.DMA,
            pltpu.SemaphoreType.DMA,
            pltpu.SemaphoreType.DMA,
            pltpu.SemaphoreType.DMA,
        ],
    )(ranks.T, suits.T, rank_table.reshape(14 * 8), suit_table.reshape(5 * 4))
    return res[:, :_L, :].transpose(2, 1, 0)


# speedup vs baseline: 117.2045x; 1.6121x over previous
import functools

import jax
import jax.numpy as jnp
from jax import lax
from jax.experimental import pallas as pl
from jax.experimental.pallas import tpu as pltpu
from jax.experimental.pallas import tpu_sc as plsc

_LANES = 16
_NUM_WORKERS = 32
_ICHUNK = 128
_L = 50
_D = 12
_CH = 6


def _card_embed_body(ranks_hbm, suits_hbm, rtab_hbm, stab_hbm, out_hbm,
                     rtab_v, stab_v, ctab_v, rk0, rk1, st0, st1, oa, ob,
                     sin0, sin1, sa, sb, n_i):
    num_cores = jax.lax.axis_size("c")
    wid = lax.axis_index("s") * num_cores + lax.axis_index("c")

    pltpu.sync_copy(rtab_hbm, rtab_v)
    pltpu.sync_copy(stab_hbm, stab_v)

    lanes = lax.iota(jnp.int32, _LANES)

    for k in range(5):
        t = jnp.minimum(lanes + _LANES * k, 69)
        r = t // 5
        s = t - r * 5
        for c in range(_D):
            if c < 8:
                vals = plsc.load_gather(rtab_v, [r * 8 + c])
            else:
                vals = plsc.load_gather(stab_v, [s * 4 + (c - 8)])
            plsc.store_scatter(ctab_v, [t * _D + c], vals)

    per_w = n_i // _NUM_WORKERS
    groups_per_j = _ICHUNK // _LANES
    nchunks = per_w // _ICHUNK

    def start_in(kc, rk, st, sem):
        i0 = wid * per_w + kc * _ICHUNK
        return (
            pltpu.async_copy(ranks_hbm.at[:, pl.ds(i0, _ICHUNK)], rk, sem),
            pltpu.async_copy(suits_hbm.at[:, pl.ds(i0, _ICHUNK)], st, sem),
        )

    def half(kc, c0, obuf, sem, rk, st):
        i0 = wid * per_w + kc * _ICHUNK

        @plsc.parallel_loop(0, _L * groups_per_j, step=1, unroll=4)
        def grp(g2):
            j = lax.shift_right_logical(g2, 3)
            g = lax.bitwise_and(g2, 7)
            col = g * _LANES
            r16 = rk[j, pl.ds(col, _LANES)]
            s16 = st[j, pl.ds(col, _LANES)]
            cidx = r16 * (5 * _D) + s16 * _D
            for cc in range(_CH):
                vals = plsc.load_gather(ctab_v, [cidx + (c0 + cc)])
                obuf[cc * _L + j, pl.ds(col, _LANES)] = vals

        return [
            pltpu.async_copy(obuf.at[pl.ds(cc * _L, _L), :],
                             out_hbm.at[c0 + cc, :, pl.ds(i0, _ICHUNK)],
                             sem)
            for cc in range(_CH)
        ]

    in_bufs = ((rk0, st0, sin0), (rk1, st1, sin1))
    in_handles = {0: start_in(0, *in_bufs[0])}
    pend = {"a": None, "b": None}

    for kc in range(nchunks):
        rk, st, _ = in_bufs[kc % 2]
        for h in in_handles.pop(kc):
            h.wait()
        if kc + 1 < nchunks:
            in_handles[kc + 1] = start_in(kc + 1, *in_bufs[(kc + 1) % 2])
        for key, c0, obuf, sem in (("a", 0, oa, sa), ("b", _CH, ob, sb)):
            if pend[key] is not None:
                for h in pend[key]:
                    h.wait()
            pend[key] = half(kc, c0, obuf, sem, rk, st)

    for key in ("a", "b"):
        for h in pend[key]:
            h.wait()


@jax.jit
def kernel(ranks, suits, rank_table, suit_table):
    B, L = ranks.shape
    per_w = B // _NUM_WORKERS
    assert per_w * _NUM_WORKERS == B and per_w % _ICHUNK == 0 and L == _L

    mesh = plsc.VectorSubcoreMesh(core_axis_name="c", subcore_axis_name="s")
    res = pl.kernel(
        functools.partial(_card_embed_body, n_i=B),
        out_type=jax.ShapeDtypeStruct((_D, _L, B), jnp.float32),
        mesh=mesh,
        compiler_params=pltpu.CompilerParams(needs_layout_passes=False),
        scratch_types=[
            pltpu.VMEM((14 * 8,), jnp.float32),
            pltpu.VMEM((5 * 4,), jnp.float32),
            pltpu.VMEM((70 * _D,), jnp.float32),
            pltpu.VMEM((_L, _ICHUNK), jnp.int32),
            pltpu.VMEM((_L, _ICHUNK), jnp.int32),
            pltpu.VMEM((_L, _ICHUNK), jnp.int32),
            pltpu.VMEM((_L, _ICHUNK), jnp.int32),
            pltpu.VMEM((_CH * _L, _ICHUNK), jnp.float32),
            pltpu.VMEM((_CH * _L, _ICHUNK), jnp.float32),
            pltpu.SemaphoreType.DMA,
            pltpu.SemaphoreType.DMA,
            pltpu.SemaphoreType.DMA,
            pltpu.SemaphoreType.DMA,
        ],
    )(ranks.T, suits.T, rank_table.reshape(14 * 8), suit_table.reshape(5 * 4))
    return res.transpose(2, 1, 0)
